# baseline (speedup 1.0000x reference)
import functools

import jax
import jax.numpy as jnp
from jax import lax
from jax.experimental import pallas as pl
from jax.experimental.pallas import tpu as pltpu
from jax.experimental.pallas import tpu_sc as plsc

N_CHUNK = 4


def _gather_call(B, D, dtype):
    info = plsc.get_sparse_core_info()
    NC, NS = info.num_cores, info.num_subcores
    NW = NC * NS
    b_per_w = B // NW
    chunk = b_per_w // N_CHUNK

    mesh = plsc.VectorSubcoreMesh(core_axis_name="c", subcore_axis_name="s")

    @functools.partial(
        pl.kernel,
        mesh=mesh,
        out_type=jax.ShapeDtypeStruct((B, D), dtype),
        scratch_types=(
            [pltpu.VMEM((N_CHUNK, chunk), jnp.int32)]
            + [pltpu.VMEM((chunk, D), dtype) for _ in range(N_CHUNK)]
            + [pltpu.SemaphoreType.DMA for _ in range(2 * N_CHUNK)]
        ),
    )
    def k(t_hbm, pe_hbm, out_hbm, idx_v, *bufs_and_sems):
        bufs = bufs_and_sems[:N_CHUNK]
        gsems = bufs_and_sems[N_CHUNK : 2 * N_CHUNK]
        osems = bufs_and_sems[2 * N_CHUNK :]
        wid = lax.axis_index("s") * NC + lax.axis_index("c")
        base = wid * b_per_w
        pltpu.sync_copy(t_hbm.at[pl.ds(wid * N_CHUNK, N_CHUNK)], idx_v)
        gathers = [
            pltpu.async_copy(pe_hbm.at[idx_v.at[j]], bufs[j], gsems[j])
            for j in range(N_CHUNK)
        ]
        outs = []
        for j in range(N_CHUNK):
            gathers[j].wait()
            outs.append(
                pltpu.async_copy(
                    bufs[j], out_hbm.at[pl.ds(base + j * chunk, chunk)], osems[j]
                )
            )
        for o in outs:
            o.wait()

    return k


def kernel(t, pe):
    t = t.astype(jnp.int32)
    if t.ndim > 1:
        t = jnp.squeeze(t, axis=-1)
    B = t.shape[0]
    D = pe.shape[1]
    info = plsc.get_sparse_core_info()
    NW = info.num_cores * info.num_subcores
    t2 = t.reshape(NW * N_CHUNK, B // (NW * N_CHUNK))
    return _gather_call(B, D, pe.dtype)(t2, pe)


# speedup vs baseline: 1.1460x; 1.1460x over previous
import functools

import jax
import jax.numpy as jnp
from jax import lax
from jax.experimental import pallas as pl
from jax.experimental.pallas import tpu as pltpu
from jax.experimental.pallas import tpu_sc as plsc

N_CHUNK = 4


def _gather_call(B, V, D, dtype):
    info = plsc.get_sparse_core_info()
    NC, NS = info.num_cores, info.num_subcores
    NW = NC * NS
    b_per_w = B // NW
    chunk = b_per_w // N_CHUNK
    stage_tiles = 5
    rows_per_stage = V // stage_tiles

    mesh = plsc.VectorSubcoreMesh(core_axis_name="c", subcore_axis_name="s")

    @functools.partial(
        pl.kernel,
        mesh=mesh,
        out_type=jax.ShapeDtypeStruct((B, D), dtype),
        scratch_types=(
            [
                pltpu.VMEM_SHARED((V, D), dtype),
                pltpu.VMEM((N_CHUNK, chunk), jnp.int32),
            ]
            + [pltpu.VMEM((chunk, D), dtype) for _ in range(N_CHUNK)]
            + [pltpu.SemaphoreType.DMA for _ in range(2 * N_CHUNK)]
        ),
    )
    def k(t_hbm, pe_hbm, out_hbm, table_sh, idx_v, *bufs_and_sems):
        bufs = bufs_and_sems[:N_CHUNK]
        gsems = bufs_and_sems[N_CHUNK : 2 * N_CHUNK]
        osems = bufs_and_sems[2 * N_CHUNK :]
        c = lax.axis_index("c")
        s = lax.axis_index("s")
        wid = s * NC + c
        base = wid * b_per_w

        @pl.when(s < stage_tiles)
        def _():
            pltpu.sync_copy(
                pe_hbm.at[pl.ds(s * rows_per_stage, rows_per_stage)],
                table_sh.at[pl.ds(s * rows_per_stage, rows_per_stage)],
            )

        pltpu.sync_copy(t_hbm.at[pl.ds(wid * N_CHUNK, N_CHUNK)], idx_v)
        plsc.subcore_barrier()

        gathers = [
            pltpu.async_copy(table_sh.at[idx_v.at[j]], bufs[j], gsems[j])
            for j in range(N_CHUNK)
        ]
        outs = []
        for j in range(N_CHUNK):
            gathers[j].wait()
            outs.append(
                pltpu.async_copy(
                    bufs[j], out_hbm.at[pl.ds(base + j * chunk, chunk)], osems[j]
                )
            )
        for o in outs:
            o.wait()

    return k


def kernel(t, pe):
    t = t.astype(jnp.int32)
    if t.ndim > 1:
        t = jnp.squeeze(t, axis=-1)
    B = t.shape[0]
    V, D = pe.shape
    info = plsc.get_sparse_core_info()
    NW = info.num_cores * info.num_subcores
    t2 = t.reshape(NW * N_CHUNK, B // (NW * N_CHUNK))
    return _gather_call(B, V, D, pe.dtype)(t2, pe)
